# Initial kernel scaffold; baseline (speedup 1.0000x reference)
import jax
import jax.numpy as jnp
from jax.experimental import pallas as pl


def kernel(a, q_dynamics, b_dynamics, e_dynamics, q_latent, b_latent, rbf, D, N, NM, params):
    raise NotImplementedError("write your pallas kernel here")


# speedup vs baseline: 6.8680x; 6.8680x over previous
import functools

import jax
import jax.numpy as jnp
from jax import lax
from jax.experimental import pallas as pl
from jax.experimental.pallas import tpu as pltpu
from jax.experimental.pallas import tpu_sc as plsc

B, A, NB, F, R = 4, 512, 32, 128, 20
E = B * A * NB
M = B * A
CUTOFF = 5.0
P = 9

NC, NS = 2, 16
NW = NC * NS
EW = E // NW
CH = 128
NCH = EW // CH
AW = M // NW


def _sigmoid(x):
    return 1.0 / (1.0 + jnp.exp(-x))


def _silu(x):
    return x * _sigmoid(x)


TB1 = 256


def _stage1_body(a_ref, qd_ref, ql_ref,
                 wa1, ba1, wa2, ba2,
                 wq1, bq1, wq2, bq2,
                 wm1, bm1, wm2, bm2,
                 we1, be1, we2, be2,
                 amsij_o, qdn_o, qln_o, emlp_o):
    x = a_ref[...]

    def mlp(w1, b1, w2, b2):
        h = _silu(jnp.dot(x, w1[...]) + b1[...])
        return jnp.dot(h, w2[...]) + b2[...]

    amsij_o[...] = mlp(wa1, ba1, wa2, ba2)
    q = mlp(wq1, bq1, wq2, bq2)
    qm = mlp(wm1, bm1, wm2, bm2)
    qdn_o[...] = qd_ref[...] + q * qm
    qln_o[...] = ql_ref[...] + q
    emlp_o[...] = mlp(we1, be1, we2, be2)


def _stage1(af, qdf, qlf, pa, pq, pqm, pe):
    n = M // TB1
    row = pl.BlockSpec((TB1, F), lambda i: (i, 0))
    col = pl.BlockSpec((TB1, 1), lambda i: (i, 0))
    wspec = lambda s: pl.BlockSpec(s, lambda i: (0, 0))
    specs_w = []
    args_w = []
    for (w1, b1, w2, b2) in (pa, pq, pqm, pe):
        args_w += [w1, b1.reshape(1, -1), w2, b2.reshape(1, -1)]
        specs_w += [wspec(w1.shape), wspec((1, b1.shape[0])),
                    wspec(w2.shape), wspec((1, b2.shape[0]))]
    return pl.pallas_call(
        _stage1_body,
        grid=(n,),
        in_specs=[row, row, col] + specs_w,
        out_specs=[row, row, col, row],
        out_shape=[
            jax.ShapeDtypeStruct((M, F), jnp.float32),
            jax.ShapeDtypeStruct((M, F), jnp.float32),
            jax.ShapeDtypeStruct((M, 1), jnp.float32),
            jax.ShapeDtypeStruct((M, F), jnp.float32),
        ],
    )(af, qdf, qlf, *args_w)


def _sc_gather_kernel(nidx, amsij, qdyn, aj_o, qj_o, idx_v, abuf, qbuf,
                      sem_a, sem_q):
    wid = lax.axis_index("c") * NS + lax.axis_index("s")
    ebase = wid * EW
    roff = (wid // (NW // B)) * A

    def body(c, carry):
        eb = ebase + c * CH
        pltpu.sync_copy(nidx.at[pl.ds(eb, CH)], idx_v)
        for i in range(CH // 16):
            sl = pl.ds(i * 16, 16)
            idx_v[sl] = idx_v[sl] + roff
        ca = pltpu.async_copy(amsij.at[idx_v], abuf, sem_a)
        cq = pltpu.async_copy(qdyn.at[idx_v], qbuf, sem_q)
        ca.wait()
        wa = pltpu.async_copy(abuf, aj_o.at[pl.ds(eb, CH)], sem_a)
        cq.wait()
        wq = pltpu.async_copy(qbuf, qj_o.at[pl.ds(eb, CH)], sem_q)
        wa.wait()
        wq.wait()
        return carry

    lax.fori_loop(0, NCH, body, 0)


@functools.cache
def _sc_gather_built():
    return functools.partial(
        pl.kernel,
        mesh=plsc.VectorSubcoreMesh(core_axis_name="c", subcore_axis_name="s"),
        out_type=[
            jax.ShapeDtypeStruct((E, F), jnp.float32),
            jax.ShapeDtypeStruct((E, F), jnp.float32),
        ],
        scratch_types=[
            pltpu.VMEM((CH,), jnp.int32),
            pltpu.VMEM((CH, F), jnp.float32),
            pltpu.VMEM((CH, F), jnp.float32),
            pltpu.SemaphoreType.DMA,
            pltpu.SemaphoreType.DMA,
        ],
    )(_sc_gather_kernel)


def _sc_gather(nflat, amsij, qdn):
    return _sc_gather_built()(nflat, amsij, qdn)


TA = 32
RE = TA * NB


def _stage2_body(amsij_ref, qdn_ref, emlp_ref, a_ref, edyn_ref,
                 aj_ref, qj_ref, rbf_ref, d_ref, bdyn_ref, blat_ref,
                 wr, br, wb1, bb1, wb2, bb2, wm1, bm1, wm2, bm2,
                 aout_o, edn_o, bdn_o, bln_o):
    dv = d_ref[...]
    x = dv * (1.0 / CUTOFF)
    x2 = x * x
    x4 = x2 * x2
    x8 = x4 * x4
    x9 = x8 * x
    x10 = x9 * x
    x11 = x10 * x
    c1 = (P + 1.0) * (P + 2.0) / 2.0
    c2 = P * (P + 2.0)
    c3 = P * (P + 1.0) / 2.0
    cut = (1.0 - c1 * x9 + c2 * x10 - c3 * x11)
    cut = cut * (dv < CUTOFF).astype(jnp.float32)

    rbfm = (jnp.dot(rbf_ref[...], wr[...]) + br[...]) * cut

    am = amsij_ref[...]
    ai = jnp.broadcast_to(am[:, None, :], (TA, NB, F)).reshape(RE, F)
    msij = ai * aj_ref[...] * rbfm

    h = _silu(jnp.dot(msij, wb1[...]) + bb1[...])
    bij = jnp.dot(h, wb2[...]) + bb2[...]
    h2 = _silu(jnp.dot(msij, wm1[...]) + bm1[...])
    m = jnp.dot(h2, wm2[...]) + bm2[...]

    bdn = bdyn_ref[...] + bij * m
    bdn_o[...] = bdn
    bln_o[...] = blat_ref[...] + bij

    dinv = jnp.where(dv > 0.0, 1.0 / dv, 0.0)
    qd = qdn_ref[...]
    qi = jnp.broadcast_to(qd[:, None, :], (TA, NB, F)).reshape(RE, F)
    contrib = dinv * (qi * qj_ref[...] - bdn)
    s = jnp.sum(contrib.reshape(TA, NB, F), axis=1)

    de = emlp_ref[...] * s
    aout_o[...] = a_ref[...] + de
    edn_o[...] = edyn_ref[...] + de


def _stage2(amsij, qdn, emlp, af, edf, aj, qj, rbff, df, bdf, blf,
            prbf, pb, pbm):
    n = M // TA
    atom = pl.BlockSpec((TA, F), lambda i: (i, 0))
    edge = pl.BlockSpec((RE, F), lambda i: (i, 0))
    edge1 = pl.BlockSpec((RE, 1), lambda i: (i, 0))
    edger = pl.BlockSpec((RE, R), lambda i: (i, 0))
    wspec = lambda s: pl.BlockSpec(s, lambda i: (0, 0))
    wr, brb = prbf
    args_w = [wr, brb.reshape(1, -1)]
    specs_w = [wspec(wr.shape), wspec((1, brb.shape[0]))]
    for (w1, b1, w2, b2) in (pb, pbm):
        args_w += [w1, b1.reshape(1, -1), w2, b2.reshape(1, -1)]
        specs_w += [wspec(w1.shape), wspec((1, b1.shape[0])),
                    wspec(w2.shape), wspec((1, b2.shape[0]))]
    return pl.pallas_call(
        _stage2_body,
        grid=(n,),
        in_specs=[atom, atom, atom, atom, atom,
                  edge, edge, edger, edge1, edge, edge1] + specs_w,
        out_specs=[atom, atom, edge, edge1],
        out_shape=[
            jax.ShapeDtypeStruct((M, F), jnp.float32),
            jax.ShapeDtypeStruct((M, F), jnp.float32),
            jax.ShapeDtypeStruct((E, F), jnp.float32),
            jax.ShapeDtypeStruct((E, 1), jnp.float32),
        ],
    )(amsij, qdn, emlp, af, edf, aj, qj, rbff, df, bdf, blf, *args_w)


def kernel(a, q_dynamics, b_dynamics, e_dynamics, q_latent, b_latent,
           rbf, D, N, NM, params):
    af = a.reshape(M, F)
    qdf = q_dynamics.reshape(M, F)
    qlf = q_latent.reshape(M, 1)
    edf = e_dynamics.reshape(M, F)
    rbff = rbf.reshape(E, R)
    df = D.reshape(E, 1)
    bdf = b_dynamics.reshape(E, F)
    blf = b_latent.reshape(E, 1)
    nflat = N.reshape(E).astype(jnp.int32)

    amsij, qdn, qln, emlp = _stage1(af, qdf, qlf,
                                    params['a'], params['q'],
                                    params['qm'], params['e'])

    aj, qj = _sc_gather(nflat, amsij, qdn)

    aout, edn, bdn, bln = _stage2(amsij, qdn, emlp, af, edf,
                                  aj, qj, rbff, df, bdf, blf,
                                  params['rbf'], params['b'], params['bm'])

    return (aout.reshape(B, A, F),
            qdn.reshape(B, A, F),
            bdn.reshape(B, A, NB, F),
            edn.reshape(B, A, F),
            qln.reshape(B, A, 1),
            bln.reshape(B, A, NB))


# speedup vs baseline: 7.1664x; 1.0435x over previous
import functools

import jax
import jax.numpy as jnp
from jax import lax
from jax.experimental import pallas as pl
from jax.experimental.pallas import tpu as pltpu
from jax.experimental.pallas import tpu_sc as plsc

B, A, NB, F, R = 4, 512, 32, 128, 20
E = B * A * NB
M = B * A
CUTOFF = 5.0
P = 9

NC, NS = 2, 16
NW = NC * NS
EW = E // NW
CH = 128
NCH = EW // CH
AW = M // NW


def _sigmoid(x):
    return 1.0 / (1.0 + jnp.exp(-x))


def _silu(x):
    return x * _sigmoid(x)


TB1 = 256


def _stage1_body(a_ref, qd_ref, ql_ref,
                 wa1, ba1, wa2, ba2,
                 wq1, bq1, wq2, bq2,
                 wm1, bm1, wm2, bm2,
                 we1, be1, we2, be2,
                 amsij_o, qdn_o, qln_o, emlp_o):
    x = a_ref[...]

    def mlp(w1, b1, w2, b2):
        h = _silu(jnp.dot(x, w1[...]) + b1[...])
        return jnp.dot(h, w2[...]) + b2[...]

    amsij_o[...] = mlp(wa1, ba1, wa2, ba2)
    q = mlp(wq1, bq1, wq2, bq2)
    qm = mlp(wm1, bm1, wm2, bm2)
    qdn_o[...] = qd_ref[...] + q * qm
    qln_o[...] = ql_ref[...] + q
    emlp_o[...] = mlp(we1, be1, we2, be2)


def _stage1(af, qdf, qlf, pa, pq, pqm, pe):
    n = M // TB1
    row = pl.BlockSpec((TB1, F), lambda i: (i, 0))
    col = pl.BlockSpec((TB1, 1), lambda i: (i, 0))
    wspec = lambda s: pl.BlockSpec(s, lambda i: (0, 0))
    specs_w = []
    args_w = []
    for (w1, b1, w2, b2) in (pa, pq, pqm, pe):
        args_w += [w1, b1.reshape(1, -1), w2, b2.reshape(1, -1)]
        specs_w += [wspec(w1.shape), wspec((1, b1.shape[0])),
                    wspec(w2.shape), wspec((1, b2.shape[0]))]
    return pl.pallas_call(
        _stage1_body,
        grid=(n,),
        in_specs=[row, row, col] + specs_w,
        out_specs=[row, row, col, row],
        out_shape=[
            jax.ShapeDtypeStruct((M, F), jnp.float32),
            jax.ShapeDtypeStruct((M, F), jnp.float32),
            jax.ShapeDtypeStruct((M, 1), jnp.float32),
            jax.ShapeDtypeStruct((M, F), jnp.float32),
        ],
    )(af, qdf, qlf, *args_w)


def _lane_bcast(v16, e):
    idx = jnp.full((16, 1), e, jnp.int32)
    return lax.gather(
        v16, idx,
        lax.GatherDimensionNumbers(offset_dims=(), collapsed_slice_dims=(0,),
                                   start_index_map=(0,)),
        (1,), mode=lax.GatherScatterMode.PROMISE_IN_BOUNDS)


def _sc_gather_kernel(nidx2, d2, amsij, qdyn, aj_o, qsum_o,
                      idx_v, w_v, ab0, ab1, qb0, qb1, qs_v,
                      sga0, sga1, sgq0, sgq1, swa0, swa1):
    wid = lax.axis_index("c") * NS + lax.axis_index("s")
    ebase = wid * EW
    abase = wid * AW
    roff = (wid // (NW // B)) * A

    pltpu.sync_copy(nidx2.at[pl.ds(wid * NCH, NCH)], idx_v)
    pltpu.sync_copy(d2.at[pl.ds(wid * NCH, NCH)], w_v)
    for r in range(NCH):
        for i in range(CH // 16):
            sl = pl.ds(i * 16, 16)
            idx_v[r, sl] = idx_v[r, sl] + roff
            dd = w_v[r, sl]
            w_v[r, sl] = jnp.where(dd > 0.0, 1.0 / dd, 0.0)

    def start(c, ab, qb, sga, sgq):
        row = idx_v.at[c]
        pltpu.async_copy(amsij.at[row], ab, sga)
        pltpu.async_copy(qdyn.at[row], qb, sgq)

    start(0, ab0, qb0, sga0, sgq0)
    start(1, ab1, qb1, sga1, sgq1)

    def section(c, ab, qb, sga, sgq, swa):
        pltpu.make_async_copy(amsij.at[idx_v.at[0]], ab, sga).wait()
        pltpu.make_async_copy(qdyn.at[idx_v.at[0]], qb, sgq).wait()
        pltpu.async_copy(ab, aj_o.at[pl.ds(ebase + c * CH, CH)], swa)
        for k in range(CH // NB):
            acc = [jnp.zeros((16,), jnp.float32) for _ in range(F // 16)]
            for g in range(NB // 16):
                w16 = w_v[c, pl.ds((k * 2 + g) * 16, 16)]
                for e in range(16):
                    we = _lane_bcast(w16, e)
                    r = k * NB + g * 16 + e
                    for f in range(F // 16):
                        acc[f] = acc[f] + qb[r, pl.ds(f * 16, 16)] * we
            for f in range(F // 16):
                qs_v[c * (CH // NB) + k, pl.ds(f * 16, 16)] = acc[f]
        @pl.when(c + 2 < NCH)
        def _():
            pltpu.make_async_copy(ab, aj_o.at[pl.ds(ebase + c * CH, CH)],
                                  swa).wait()
            start(c + 2, ab, qb, sga, sgq)

    def body(co, carry):
        section(2 * co, ab0, qb0, sga0, sgq0, swa0)
        section(2 * co + 1, ab1, qb1, sga1, sgq1, swa1)
        return carry

    lax.fori_loop(0, NCH // 2, body, 0)
    pltpu.make_async_copy(ab0, aj_o.at[pl.ds(ebase + (NCH - 2) * CH, CH)],
                          swa0).wait()
    pltpu.make_async_copy(ab1, aj_o.at[pl.ds(ebase + (NCH - 1) * CH, CH)],
                          swa1).wait()
    pltpu.sync_copy(qs_v, qsum_o.at[pl.ds(abase, AW)])


@functools.cache
def _sc_gather_built():
    return functools.partial(
        pl.kernel,
        mesh=plsc.VectorSubcoreMesh(core_axis_name="c", subcore_axis_name="s"),
        out_type=[
            jax.ShapeDtypeStruct((E, F), jnp.float32),
            jax.ShapeDtypeStruct((M, F), jnp.float32),
        ],
        scratch_types=[
            pltpu.VMEM((NCH, CH), jnp.int32),
            pltpu.VMEM((NCH, CH), jnp.float32),
            pltpu.VMEM((CH, F), jnp.float32),
            pltpu.VMEM((CH, F), jnp.float32),
            pltpu.VMEM((CH, F), jnp.float32),
            pltpu.VMEM((CH, F), jnp.float32),
            pltpu.VMEM((AW, F), jnp.float32),
        ] + [pltpu.SemaphoreType.DMA] * 6,
    )(_sc_gather_kernel)


def _sc_gather(nidx2, d2, amsij, qdn):
    return _sc_gather_built()(nidx2, d2, amsij, qdn)


TA = 32
RE = TA * NB


def _stage2_body(amsij_ref, qdn_ref, emlp_ref, a_ref, edyn_ref, qsum_ref,
                 aj_ref, rbf_ref, d_ref, bdyn_ref, blat_ref,
                 wr, br, wb1, bb1, wb2, bb2, wm1, bm1, wm2, bm2,
                 aout_o, edn_o, bdn_o, bln_o):
    dv = d_ref[...]
    x = dv * (1.0 / CUTOFF)
    x2 = x * x
    x4 = x2 * x2
    x8 = x4 * x4
    x9 = x8 * x
    x10 = x9 * x
    x11 = x10 * x
    c1 = (P + 1.0) * (P + 2.0) / 2.0
    c2 = P * (P + 2.0)
    c3 = P * (P + 1.0) / 2.0
    cut = (1.0 - c1 * x9 + c2 * x10 - c3 * x11)
    cut = cut * (dv < CUTOFF).astype(jnp.float32)

    rbfm = (jnp.dot(rbf_ref[...], wr[...]) + br[...]) * cut

    am = amsij_ref[...]
    ai = jnp.broadcast_to(am[:, None, :], (TA, NB, F)).reshape(RE, F)
    msij = ai * aj_ref[...] * rbfm

    h = _silu(jnp.dot(msij, wb1[...]) + bb1[...])
    bij = jnp.dot(h, wb2[...]) + bb2[...]
    h2 = _silu(jnp.dot(msij, wm1[...]) + bm1[...])
    m = jnp.dot(h2, wm2[...]) + bm2[...]

    bdn = bdyn_ref[...] + bij * m
    bdn_o[...] = bdn
    bln_o[...] = blat_ref[...] + bij

    dinv = jnp.where(dv > 0.0, 1.0 / dv, 0.0)
    sb = jnp.sum((dinv * bdn).reshape(TA, NB, F), axis=1)
    de = emlp_ref[...] * (qdn_ref[...] * qsum_ref[...] - sb)
    aout_o[...] = a_ref[...] + de
    edn_o[...] = edyn_ref[...] + de


def _stage2(amsij, qdn, emlp, af, edf, qsum, aj, rbff, df, bdf, blf,
            prbf, pb, pbm):
    n = M // TA
    atom = pl.BlockSpec((TA, F), lambda i: (i, 0))
    edge = pl.BlockSpec((RE, F), lambda i: (i, 0))
    edge1 = pl.BlockSpec((RE, 1), lambda i: (i, 0))
    edger = pl.BlockSpec((RE, R), lambda i: (i, 0))
    wspec = lambda s: pl.BlockSpec(s, lambda i: (0, 0))
    wr, brb = prbf
    args_w = [wr, brb.reshape(1, -1)]
    specs_w = [wspec(wr.shape), wspec((1, brb.shape[0]))]
    for (w1, b1, w2, b2) in (pb, pbm):
        args_w += [w1, b1.reshape(1, -1), w2, b2.reshape(1, -1)]
        specs_w += [wspec(w1.shape), wspec((1, b1.shape[0])),
                    wspec(w2.shape), wspec((1, b2.shape[0]))]
    return pl.pallas_call(
        _stage2_body,
        grid=(n,),
        in_specs=[atom, atom, atom, atom, atom, atom,
                  edge, edger, edge1, edge, edge1] + specs_w,
        out_specs=[atom, atom, edge, edge1],
        out_shape=[
            jax.ShapeDtypeStruct((M, F), jnp.float32),
            jax.ShapeDtypeStruct((M, F), jnp.float32),
            jax.ShapeDtypeStruct((E, F), jnp.float32),
            jax.ShapeDtypeStruct((E, 1), jnp.float32),
        ],
    )(amsij, qdn, emlp, af, edf, qsum, aj, rbff, df, bdf, blf, *args_w)


def kernel(a, q_dynamics, b_dynamics, e_dynamics, q_latent, b_latent,
           rbf, D, N, NM, params):
    af = a.reshape(M, F)
    qdf = q_dynamics.reshape(M, F)
    qlf = q_latent.reshape(M, 1)
    edf = e_dynamics.reshape(M, F)
    rbff = rbf.reshape(E, R)
    df = D.reshape(E, 1)
    bdf = b_dynamics.reshape(E, F)
    blf = b_latent.reshape(E, 1)
    nidx2 = N.reshape(E // CH, CH).astype(jnp.int32)
    d2 = D.reshape(E // CH, CH)

    amsij, qdn, qln, emlp = _stage1(af, qdf, qlf,
                                    params['a'], params['q'],
                                    params['qm'], params['e'])

    aj, qsum = _sc_gather(nidx2, d2, amsij, qdn)

    aout, edn, bdn, bln = _stage2(amsij, qdn, emlp, af, edf,
                                  qsum, aj, rbff, df, bdf, blf,
                                  params['rbf'], params['b'], params['bm'])

    return (aout.reshape(B, A, F),
            qdn.reshape(B, A, F),
            bdn.reshape(B, A, NB, F),
            edn.reshape(B, A, F),
            qln.reshape(B, A, 1),
            bln.reshape(B, A, NB))


# speedup vs baseline: 9.6370x; 1.3447x over previous
import functools

import jax
import jax.numpy as jnp
from jax import lax
from jax.experimental import pallas as pl
from jax.experimental.pallas import tpu as pltpu
from jax.experimental.pallas import tpu_sc as plsc

B, A, NB, F, R = 4, 512, 32, 128, 20
E = B * A * NB
M = B * A
CUTOFF = 5.0
P = 9

NC, NS = 2, 16
NW = NC * NS
EW = E // NW
CH = 128
NCH = EW // CH
AW = M // NW


def _sigmoid(x):
    return 1.0 / (1.0 + jnp.exp(-x))


def _silu(x):
    return x * _sigmoid(x)


TB1 = 256


def _stage1_body(a_ref, qd_ref, ql_ref,
                 wa1, ba1, wa2, ba2,
                 wq1, bq1, wq2, bq2,
                 wm1, bm1, wm2, bm2,
                 we1, be1, we2, be2,
                 amsij_o, qdn_o, qln_o, emlp_o):
    x = a_ref[...]

    def mlp(w1, b1, w2, b2):
        h = _silu(jnp.dot(x, w1[...]) + b1[...])
        return jnp.dot(h, w2[...]) + b2[...]

    amsij_o[...] = mlp(wa1, ba1, wa2, ba2)
    q = mlp(wq1, bq1, wq2, bq2)
    qm = mlp(wm1, bm1, wm2, bm2)
    qdn_o[...] = qd_ref[...] + q * qm
    qln_o[...] = ql_ref[...] + q
    emlp_o[...] = mlp(we1, be1, we2, be2)


def _stage1(af, qdf, qlf, pa, pq, pqm, pe):
    n = M // TB1
    row = pl.BlockSpec((TB1, F), lambda i: (i, 0))
    col = pl.BlockSpec((TB1, 1), lambda i: (i, 0))
    wspec = lambda s: pl.BlockSpec(s, lambda i: (0, 0))
    specs_w = []
    args_w = []
    for (w1, b1, w2, b2) in (pa, pq, pqm, pe):
        args_w += [w1, b1.reshape(1, -1), w2, b2.reshape(1, -1)]
        specs_w += [wspec(w1.shape), wspec((1, b1.shape[0])),
                    wspec(w2.shape), wspec((1, b2.shape[0]))]
    return pl.pallas_call(
        _stage1_body,
        grid=(n,),
        in_specs=[row, row, col] + specs_w,
        out_specs=[row, row, col, row],
        out_shape=[
            jax.ShapeDtypeStruct((M, F), jnp.float32),
            jax.ShapeDtypeStruct((M, F), jnp.float32),
            jax.ShapeDtypeStruct((M, 1), jnp.float32),
            jax.ShapeDtypeStruct((M, F), jnp.float32),
        ],
    )(af, qdf, qlf, *args_w)


def _lane_bcast(v16, e):
    idx = jnp.full((16, 1), e, jnp.int32)
    return lax.gather(
        v16, idx,
        lax.GatherDimensionNumbers(offset_dims=(), collapsed_slice_dims=(0,),
                                   start_index_map=(0,)),
        (1,), mode=lax.GatherScatterMode.PROMISE_IN_BOUNDS)


def _sc_gather_kernel(nidx2, d2, amsij, qdyn, aj_o, qsum_o,
                      idx_v, w_v, ab0, ab1, qb0, qb1, qs_v,
                      sga0, sga1, sgq0, sgq1, swa0, swa1):
    wid = lax.axis_index("c") * NS + lax.axis_index("s")
    ebase = wid * EW
    abase = wid * AW
    roff = (wid // (NW // B)) * A

    pltpu.sync_copy(nidx2.at[pl.ds(wid * NCH, NCH)], idx_v)
    pltpu.sync_copy(d2.at[pl.ds(wid * NCH, NCH)], w_v)
    for r in range(NCH):
        for i in range(CH // 16):
            sl = pl.ds(i * 16, 16)
            idx_v[r, sl] = idx_v[r, sl] + roff
            dd = w_v[r, sl]
            w_v[r, sl] = jnp.where(dd > 0.0, 1.0 / dd, 0.0)

    def start(c, ab, qb, sga, sgq):
        row = idx_v.at[c]
        pltpu.async_copy(amsij.at[row], ab, sga)
        pltpu.async_copy(qdyn.at[row], qb, sgq)

    start(0, ab0, qb0, sga0, sgq0)
    start(1, ab1, qb1, sga1, sgq1)

    def section(c, ab, qb, sga, sgq, swa):
        pltpu.make_async_copy(amsij.at[idx_v.at[0]], ab, sga).wait()
        pltpu.make_async_copy(qdyn.at[idx_v.at[0]], qb, sgq).wait()
        pltpu.async_copy(ab, aj_o.at[pl.ds(ebase + c * CH, CH)], swa)
        for k in range(CH // NB):
            acc = [jnp.zeros((16,), jnp.float32) for _ in range(F // 16)]
            for g in range(NB // 16):
                w16 = w_v[c, pl.ds((k * 2 + g) * 16, 16)]
                for e in range(16):
                    we = _lane_bcast(w16, e)
                    r = k * NB + g * 16 + e
                    for f in range(F // 16):
                        acc[f] = acc[f] + qb[r, pl.ds(f * 16, 16)] * we
            for f in range(F // 16):
                qs_v[c * (CH // NB) + k, pl.ds(f * 16, 16)] = acc[f]
        @pl.when(c + 2 < NCH)
        def _():
            pltpu.make_async_copy(ab, aj_o.at[pl.ds(ebase + c * CH, CH)],
                                  swa).wait()
            start(c + 2, ab, qb, sga, sgq)

    def body(co, carry):
        section(2 * co, ab0, qb0, sga0, sgq0, swa0)
        section(2 * co + 1, ab1, qb1, sga1, sgq1, swa1)
        return carry

    lax.fori_loop(0, NCH // 2, body, 0)
    pltpu.make_async_copy(ab0, aj_o.at[pl.ds(ebase + (NCH - 2) * CH, CH)],
                          swa0).wait()
    pltpu.make_async_copy(ab1, aj_o.at[pl.ds(ebase + (NCH - 1) * CH, CH)],
                          swa1).wait()
    pltpu.sync_copy(qs_v, qsum_o.at[pl.ds(abase, AW)])


@functools.cache
def _sc_gather_built():
    return functools.partial(
        pl.kernel,
        mesh=plsc.VectorSubcoreMesh(core_axis_name="c", subcore_axis_name="s"),
        out_type=[
            jax.ShapeDtypeStruct((E, F), jnp.float32),
            jax.ShapeDtypeStruct((M, F), jnp.float32),
        ],
        scratch_types=[
            pltpu.VMEM((NCH, CH), jnp.int32),
            pltpu.VMEM((NCH, CH), jnp.float32),
            pltpu.VMEM((CH, F), jnp.float32),
            pltpu.VMEM((CH, F), jnp.float32),
            pltpu.VMEM((CH, F), jnp.float32),
            pltpu.VMEM((CH, F), jnp.float32),
            pltpu.VMEM((AW, F), jnp.float32),
        ] + [pltpu.SemaphoreType.DMA] * 6,
    )(_sc_gather_kernel)


def _sc_gather(nidx2, d2, amsij, qdn):
    return _sc_gather_built()(nidx2, d2, amsij, qdn)


TA = 32
RE = TA * NB


def _stage2_body(amsij_ref, qdn_ref, emlp_ref, a_ref, edyn_ref, qsum_ref,
                 aj_ref, rbf_ref, d_ref, bdyn_ref, blat_ref,
                 wr, br, wb1, bb1, wb2, bb2, wb2r, wm1, bm1, wm2, bm2,
                 aout_o, edn_o, bdn_o, bln_o):
    lane = lax.broadcasted_iota(jnp.int32, (RE, NB), 1)
    row = lax.broadcasted_iota(jnp.int32, (RE, NB), 0)
    sel = (lane == row % NB).astype(jnp.float32)

    def expand_col(x_an):
        z = jnp.broadcast_to(x_an[:, None, :], (TA, NB, NB)).reshape(RE, NB)
        return jnp.sum(z * sel, axis=1, keepdims=True)

    dv = expand_col(d_ref[...])
    x = dv * (1.0 / CUTOFF)
    x2 = x * x
    x4 = x2 * x2
    x8 = x4 * x4
    x9 = x8 * x
    x10 = x9 * x
    x11 = x10 * x
    c1 = (P + 1.0) * (P + 2.0) / 2.0
    c2 = P * (P + 2.0)
    c3 = P * (P + 1.0) / 2.0
    cut = (1.0 - c1 * x9 + c2 * x10 - c3 * x11)
    cut = cut * (dv < CUTOFF).astype(jnp.float32)

    rbfm = (jnp.dot(rbf_ref[...], wr[...]) + br[...]) * cut

    am = amsij_ref[...]
    ai = jnp.broadcast_to(am[:, None, :], (TA, NB, F)).reshape(RE, F)
    msij = ai * aj_ref[...] * rbfm

    h = _silu(jnp.dot(msij, wb1[...]) + bb1[...])
    bij = jnp.dot(h, wb2[...]) + bb2[...]
    h2 = _silu(jnp.dot(msij, wm1[...]) + bm1[...])
    m = jnp.dot(h2, wm2[...]) + bm2[...]

    bdn = bdyn_ref[...] + bij * m
    bdn_o[...] = bdn
    bij_an = (jnp.sum(h.reshape(TA, NB, F) * wb2r[...].reshape(1, 1, F),
                      axis=2) + bb2[...])
    bln_o[...] = blat_ref[...] + bij_an

    dinv = jnp.where(dv > 0.0, 1.0 / dv, 0.0)
    sb = jnp.sum((dinv * bdn).reshape(TA, NB, F), axis=1)
    de = emlp_ref[...] * (qdn_ref[...] * qsum_ref[...] - sb)
    aout_o[...] = a_ref[...] + de
    edn_o[...] = edyn_ref[...] + de


def _stage2(amsij, qdn, emlp, af, edf, qsum, aj, rbff, df, bdf, blf,
            prbf, pb, pbm):
    n = M // TA
    atom = pl.BlockSpec((TA, F), lambda i: (i, 0))
    atomnb = pl.BlockSpec((TA, NB), lambda i: (i, 0))
    edge = pl.BlockSpec((RE, F), lambda i: (i, 0))
    edger = pl.BlockSpec((RE, R), lambda i: (i, 0))
    wspec = lambda s: pl.BlockSpec(s, lambda i: (0, 0))
    wr, brb = prbf
    args_w = [wr, brb.reshape(1, -1)]
    specs_w = [wspec(wr.shape), wspec((1, brb.shape[0]))]
    for (w1, b1, w2, b2) in (pb, pbm):
        args_w += [w1, b1.reshape(1, -1), w2, b2.reshape(1, -1)]
        specs_w += [wspec(w1.shape), wspec((1, b1.shape[0])),
                    wspec(w2.shape), wspec((1, b2.shape[0]))]
    wb2r = pb[2].reshape(-1)[None, :]
    args_w.insert(6, wb2r)
    specs_w.insert(6, wspec(wb2r.shape))
    return pl.pallas_call(
        _stage2_body,
        grid=(n,),
        in_specs=[atom, atom, atom, atom, atom, atom,
                  edge, edger, atomnb, edge, atomnb] + specs_w,
        out_specs=[atom, atom, edge, atomnb],
        out_shape=[
            jax.ShapeDtypeStruct((M, F), jnp.float32),
            jax.ShapeDtypeStruct((M, F), jnp.float32),
            jax.ShapeDtypeStruct((E, F), jnp.float32),
            jax.ShapeDtypeStruct((M, NB), jnp.float32),
        ],
    )(amsij, qdn, emlp, af, edf, qsum, aj, rbff, df, bdf, blf, *args_w)


def kernel(a, q_dynamics, b_dynamics, e_dynamics, q_latent, b_latent,
           rbf, D, N, NM, params):
    af = a.reshape(M, F)
    qdf = q_dynamics.reshape(M, F)
    qlf = q_latent.reshape(M, 1)
    edf = e_dynamics.reshape(M, F)
    rbff = rbf.reshape(E, R)
    df = D.reshape(M, NB)
    bdf = b_dynamics.reshape(E, F)
    blf = b_latent.reshape(M, NB)
    nidx2 = N.reshape(E // CH, CH).astype(jnp.int32)
    d2 = D.reshape(E // CH, CH)

    amsij, qdn, qln, emlp = _stage1(af, qdf, qlf,
                                    params['a'], params['q'],
                                    params['qm'], params['e'])

    aj, qsum = _sc_gather(nidx2, d2, amsij, qdn)

    aout, edn, bdn, bln = _stage2(amsij, qdn, emlp, af, edf,
                                  qsum, aj, rbff, df, bdf, blf,
                                  params['rbf'], params['b'], params['bm'])

    return (aout.reshape(B, A, F),
            qdn.reshape(B, A, F),
            bdn.reshape(B, A, NB, F),
            edn.reshape(B, A, F),
            qln.reshape(B, A, 1),
            bln.reshape(B, A, NB))
